# baseline (speedup 1.0000x reference)
from functools import partial

import numpy as np
import jax
import jax.numpy as jnp
from jax.experimental import pallas as pl
from jax.experimental.pallas import tpu as pltpu

_ROTS = ((13, 15, 26, 6), (17, 29, 16, 24))
_TINY = np.float32(np.finfo(np.float32).tiny)
_KS = (np.uint32(0), np.uint32(42), np.uint32(0 ^ 42 ^ 0x1BD11BDA))


def _threefry_bits(x1):
    x0 = x1
    x1 = ((x1 << np.uint32(13)) | (x1 >> np.uint32(19))) ^ x0
    for g in range(5):
        rots = _ROTS[g % 2]
        for r in rots[1:] if g == 0 else rots:
            x0 = x0 + x1
            x1 = ((x1 << np.uint32(r)) | (x1 >> np.uint32(32 - r))) ^ x0
        c0 = np.uint32(_KS[(g + 1) % 3])
        c1 = np.uint32((int(_KS[(g + 2) % 3]) + g + 1) & 0xFFFFFFFF)
        if c0:
            x0 = x0 + c0
        x1 = x1 + c1
    return x0 ^ x1


def _sampler_body(
    scal_ref, hm_ref, side_ref, out_ref, logit_ref, v_ref, *, rows_per_blk,
    ncols, nrows, nsamp, chunk
):
    blk = pl.program_id(0)
    nchunks = ncols // chunk

    @pl.when(blk < scal_ref[1])
    def _compute():
        _sampler_compute(
            scal_ref, hm_ref, side_ref, out_ref, logit_ref, v_ref, blk,
            rows_per_blk=rows_per_blk, ncols=ncols, nrows=nrows,
            nsamp=nsamp, chunk=chunk, nchunks=nchunks,
        )


def _sampler_compute(
    scal_ref, hm_ref, side_ref, out_ref, logit_ref, v_ref, blk, *,
    rows_per_blk, ncols, nrows, nsamp, chunk, nchunks
):
    use_side = blk >= scal_ref[2]
    row0 = jnp.where(
        use_side,
        scal_ref[3] + (blk - scal_ref[2]) * rows_per_blk,
        scal_ref[0] + blk * rows_per_blk,
    )

    tile = jnp.where(use_side, side_ref[...], hm_ref[...])
    hp = jnp.where(tile < 0.0, jnp.float32(0.0), tile)
    total = jnp.sum(hp, axis=1, keepdims=True)
    hp = jnp.where(total <= 0.0, jnp.float32(0.001), hp)
    logit_ref[...] = jnp.where(
        hp > 0.0,
        jnp.log(jnp.maximum(hp, jnp.float32(1e-30))),
        jnp.float32(-jnp.inf),
    )

    row_iota = jax.lax.broadcasted_iota(jnp.uint32, (rows_per_blk, chunk), 0)
    col_iota = jax.lax.broadcasted_iota(jnp.uint32, (rows_per_blk, chunk), 1)
    rc0 = row_iota * np.uint32(ncols) + col_iota
    base = jnp.uint32(row0) * np.uint32(ncols) + np.uint32(42)

    colf = jax.lax.broadcasted_iota(
        jnp.int32, (rows_per_blk, ncols), 1
    ).astype(jnp.float32)

    for s in range(nsamp):
        run_max = None
        for c in range(nchunks):
            off = base + np.uint32(s * nrows * ncols + c * chunk)
            bits = _threefry_bits(rc0 + off)
            mant = (bits >> np.uint32(9)).astype(jnp.int32)
            f = mant.astype(jnp.float32) * np.float32(2.0**-23)
            u = jnp.maximum(f, _TINY)
            v = logit_ref[:, c * chunk : (c + 1) * chunk] - jnp.log(-jnp.log(u))
            v_ref[:, c * chunk : (c + 1) * chunk] = v
            run_max = v if run_max is None else jnp.maximum(run_max, v)
        m = jnp.max(run_max, axis=1, keepdims=True)
        vall = v_ref[...]
        cand = jnp.where(vall == m, colf, jnp.float32(ncols))
        out_ref[:, s : s + 1] = jnp.min(cand, axis=1, keepdims=True)


def kernel(heatmap, num_samples):
    b, j, w, h = heatmap.shape
    ns = 16
    nrows = b * j
    ncols = w * h
    rows_per_blk = 16
    chunk = 1024
    assert nrows % rows_per_blk == 0 and ncols % chunk == 0

    flat = heatmap.reshape(nrows, ncols)
    nblk = nrows // rows_per_blk

    devices = jax.devices()
    ndev = 2 if (len(devices) >= 2 and nblk % 2 == 0) else 1
    offload = min(12, nblk // 2 - 1) if ndev == 2 else 0
    offload = max(offload, 0)

    def _pcall(scal, flat_local, side_local, grid_blocks, a_blocks):
        return pl.pallas_call(
            partial(
                _sampler_body,
                rows_per_blk=rows_per_blk,
                ncols=ncols,
                nrows=nrows,
                nsamp=ns,
                chunk=chunk,
            ),
            grid_spec=pltpu.PrefetchScalarGridSpec(
                num_scalar_prefetch=1,
                grid=(grid_blocks,),
                in_specs=[
                    pl.BlockSpec(
                        (rows_per_blk, ncols),
                        lambda i, s, ab=a_blocks: (jnp.minimum(i, ab - 1), 0),
                    ),
                    pl.BlockSpec(
                        (rows_per_blk, ncols),
                        lambda i, s, ab=a_blocks: (jnp.maximum(i - ab, 0), 0),
                    ),
                ],
                out_specs=pl.BlockSpec((rows_per_blk, ns), lambda i, s: (i, 0)),
                scratch_shapes=[
                    pltpu.VMEM((rows_per_blk, ncols), jnp.float32),
                    pltpu.VMEM((rows_per_blk, ncols), jnp.float32),
                ],
            ),
            out_shape=jax.ShapeDtypeStruct(
                (grid_blocks * rows_per_blk, ns), jnp.float32
            ),
        )(scal, flat_local, side_local)

    if ndev == 2:
        half_blks = nblk // 2
        own0 = half_blks - offload
        side_row0 = own0 * rows_per_blk
        local_rows = nrows // 2
        side = flat[side_row0 : half_blks * rows_per_blk]
        grid_blocks = half_blks + offload
        grid_rows = grid_blocks * rows_per_blk

        def _local(fl, side_l):
            ai = jax.lax.axis_index("d")
            a_base = (ai * local_rows).astype(jnp.int32)
            myblks = jnp.where(ai == 0, own0, grid_blocks).astype(jnp.int32)
            ablocks = jnp.where(ai == 0, grid_blocks, half_blks).astype(
                jnp.int32
            )
            scal = jnp.stack([a_base, myblks, ablocks, jnp.int32(side_row0)])
            return _pcall(scal, fl, side_l, grid_blocks, half_blks)

        mesh = jax.sharding.Mesh(np.array(devices[:2]), ("d",))
        P = jax.sharding.PartitionSpec
        g = jax.shard_map(
            _local,
            mesh=mesh,
            in_specs=(P("d", None), P(None, None)),
            out_specs=P("d", None),
            check_vma=False,
        )(flat, side)
        samples = jnp.concatenate(
            [
                g[:side_row0],
                g[grid_rows + local_rows : 2 * grid_rows],
                g[grid_rows : grid_rows + local_rows],
            ],
            axis=0,
        )
    else:
        scal = jnp.array([0, nblk, nblk + 1, 0], jnp.int32)
        samples = _pcall(scal, flat, flat[:rows_per_blk], nblk, nblk)

    samples = samples.reshape(b, j, ns)
    x = jnp.mod(samples, float(h))
    y = jnp.floor(samples / float(h))
    x = (x - 0.5 * h) / h
    y = (y - 0.5 * h) / h
    joint = jnp.stack((x, y), axis=-2)
    joint = jnp.transpose(joint, (0, 3, 1, 2)).reshape(b, ns, j * 2)
    joint = joint + jnp.asarray(num_samples - ns).astype(joint.dtype)
    return joint


# speedup vs baseline: 1.1395x; 1.1395x over previous
from functools import partial

import numpy as np
import jax
import jax.numpy as jnp
from jax.experimental import pallas as pl
from jax.experimental.pallas import tpu as pltpu

_ROTS = ((13, 15, 26, 6), (17, 29, 16, 24))
_TINY = np.float32(np.finfo(np.float32).tiny)
_KS = (np.uint32(0), np.uint32(42), np.uint32(0 ^ 42 ^ 0x1BD11BDA))


def _threefry_bits(x1):
    x0 = x1
    x1 = ((x1 << np.uint32(13)) | (x1 >> np.uint32(19))) ^ x0
    for g in range(5):
        rots = _ROTS[g % 2]
        for r in rots[1:] if g == 0 else rots:
            x0 = x0 + x1
            x1 = ((x1 << np.uint32(r)) | (x1 >> np.uint32(32 - r))) ^ x0
        c0 = np.uint32(_KS[(g + 1) % 3])
        c1 = np.uint32((int(_KS[(g + 2) % 3]) + g + 1) & 0xFFFFFFFF)
        if c0:
            x0 = x0 + c0
        x1 = x1 + c1
    return x0 ^ x1


def _sampler_body(
    scal_ref, hm_ref, out_ref, logit_ref, v_ref, *, rows_per_blk, ncols, nrows,
    nsamp, chunk
):
    blk = pl.program_id(0)
    nchunks = ncols // chunk

    @pl.when(blk < scal_ref[1])
    def _compute():
        _sampler_compute(
            scal_ref, hm_ref, out_ref, logit_ref, v_ref, blk,
            rows_per_blk=rows_per_blk, ncols=ncols, nrows=nrows,
            nsamp=nsamp, chunk=chunk, nchunks=nchunks,
        )


def _sampler_compute(
    scal_ref, hm_ref, out_ref, logit_ref, v_ref, blk, *, rows_per_blk, ncols,
    nrows, nsamp, chunk, nchunks
):
    tile = hm_ref[...]
    hp = jnp.where(tile < 0.0, jnp.float32(0.0), tile)
    total = jnp.sum(hp, axis=1, keepdims=True)
    hp = jnp.where(total <= 0.0, jnp.float32(0.001), hp)
    logit_ref[...] = jnp.where(
        hp > 0.0,
        jnp.log(jnp.maximum(hp, jnp.float32(1e-30))),
        jnp.float32(-jnp.inf),
    )

    row_iota = jax.lax.broadcasted_iota(jnp.uint32, (rows_per_blk, chunk), 0)
    col_iota = jax.lax.broadcasted_iota(jnp.uint32, (rows_per_blk, chunk), 1)
    rc0 = row_iota * np.uint32(ncols) + col_iota
    base = (
        (jnp.uint32(scal_ref[0]) + jnp.uint32(blk) * np.uint32(rows_per_blk))
        * np.uint32(ncols)
        + np.uint32(42)
    )

    colf = jax.lax.broadcasted_iota(
        jnp.int32, (rows_per_blk, ncols), 1
    ).astype(jnp.float32)

    for s in range(nsamp):
        run_max = None
        for c in range(nchunks):
            off = base + np.uint32(s * nrows * ncols + c * chunk)
            bits = _threefry_bits(rc0 + off)
            mant = (bits >> np.uint32(9)).astype(jnp.int32)
            f = mant.astype(jnp.float32) * np.float32(2.0**-23)
            u = jnp.maximum(f, _TINY)
            v = logit_ref[:, c * chunk : (c + 1) * chunk] - jnp.log(-jnp.log(u))
            v_ref[:, c * chunk : (c + 1) * chunk] = v
            run_max = v if run_max is None else jnp.maximum(run_max, v)
        m = jnp.max(run_max, axis=1, keepdims=True)
        vall = v_ref[...]
        cand = jnp.where(vall == m, colf, jnp.float32(ncols))
        out_ref[:, s : s + 1] = jnp.min(cand, axis=1, keepdims=True)


def kernel(heatmap, num_samples):
    b, j, w, h = heatmap.shape
    ns = 16
    nrows = b * j
    ncols = w * h
    rows_per_blk = 16
    chunk = 1024
    assert nrows % rows_per_blk == 0 and ncols % chunk == 0

    flat = heatmap.reshape(nrows, ncols)

    devices = jax.devices()
    ndev = len(devices)
    while nrows % (ndev * rows_per_blk):
        ndev -= 1
    local_rows = nrows // ndev

    def _local(flat_local, ai):
        row0 = (ai * local_rows).astype(jnp.int32)
        myblks = jnp.int32(local_rows // rows_per_blk)
        scal = jnp.stack([row0, myblks])
        return pl.pallas_call(
            partial(
                _sampler_body,
                rows_per_blk=rows_per_blk,
                ncols=ncols,
                nrows=nrows,
                nsamp=ns,
                chunk=chunk,
            ),
            grid_spec=pltpu.PrefetchScalarGridSpec(
                num_scalar_prefetch=1,
                grid=(local_rows // rows_per_blk,),
                in_specs=[
                    pl.BlockSpec((rows_per_blk, ncols), lambda i, s: (i, 0))
                ],
                out_specs=pl.BlockSpec((rows_per_blk, ns), lambda i, s: (i, 0)),
                scratch_shapes=[
                    pltpu.VMEM((rows_per_blk, ncols), jnp.float32),
                    pltpu.VMEM((rows_per_blk, ncols), jnp.float32),
                ],
            ),
            out_shape=jax.ShapeDtypeStruct((local_rows, ns), jnp.float32),
        )(scal, flat_local)

    if ndev > 1:
        mesh = jax.sharding.Mesh(np.array(devices[:ndev]), ("d",))
        P = jax.sharding.PartitionSpec
        samples = jax.shard_map(
            lambda fl: _local(fl, jax.lax.axis_index("d")),
            mesh=mesh,
            in_specs=P("d", None),
            out_specs=P("d", None),
            check_vma=False,
        )(flat)
    else:
        samples = _local(flat, jnp.int32(0))

    samples = samples.reshape(b, j, ns)
    x = jnp.mod(samples, float(h))
    y = jnp.floor(samples / float(h))
    x = (x - 0.5 * h) / h
    y = (y - 0.5 * h) / h
    joint = jnp.stack((x, y), axis=-2)
    joint = jnp.transpose(joint, (0, 3, 1, 2)).reshape(b, ns, j * 2)
    joint = joint + jnp.asarray(num_samples - ns).astype(joint.dtype)
    return joint
